# baseline (speedup 1.0000x reference)
import jax
import jax.numpy as jnp
from jax.experimental import pallas as pl
from jax.experimental.pallas import tpu as pltpu

_MAX_TILES = 4
_HIDDEN = 1280
_PATCHES = 1025
_PBLK = 208


def _body(ids_ref, gate_ref, hid_ref, emb_ref, out_ref):
    b = pl.program_id(0)
    t = pl.program_id(1)
    g = jnp.tanh(gate_ref[0])
    row = emb_ref[ids_ref[b], t]
    out_ref[...] = hid_ref[...] + (row * g).reshape(1, 1, 1, _HIDDEN)


def kernel(hidden_state, aspect_ratio_ids, embedding_table, gate):
    batch = hidden_state.shape[0]
    ids = aspect_ratio_ids.astype(jnp.int32)
    table = embedding_table.reshape(-1, _MAX_TILES, 1, _HIDDEN)
    n_rows = table.shape[0]
    grid = (batch, _MAX_TILES, pl.cdiv(_PATCHES, _PBLK))

    out = pl.pallas_call(
        _body,
        grid_spec=pltpu.PrefetchScalarGridSpec(
            num_scalar_prefetch=2,
            grid=grid,
            in_specs=[
                pl.BlockSpec(
                    (1, 1, _PBLK, _HIDDEN),
                    lambda b, t, p, ids, gate: (b, t, p, 0),
                ),
                pl.BlockSpec(
                    (n_rows, _MAX_TILES, 1, _HIDDEN),
                    lambda b, t, p, ids, gate: (0, 0, 0, 0),
                ),
            ],
            out_specs=pl.BlockSpec(
                (1, 1, _PBLK, _HIDDEN),
                lambda b, t, p, ids, gate: (b, t, p, 0),
            ),
        ),
        out_shape=jax.ShapeDtypeStruct(hidden_state.shape, hidden_state.dtype),
        compiler_params=pltpu.CompilerParams(
            dimension_semantics=("parallel", "parallel", "parallel"),
        ),
    )(ids, gate, hidden_state, table)
    return out
